# Initial kernel scaffold; baseline (speedup 1.0000x reference)
import jax
import jax.numpy as jnp
from jax.experimental import pallas as pl


def kernel(x, sigma, c, centroids_idx, W1, b1, W2, b2, W3, b3):
    raise NotImplementedError("write your pallas kernel here")


# speedup vs baseline: 55.5323x; 55.5323x over previous
import functools

import jax
import jax.numpy as jnp
from jax.experimental import pallas as pl
from jax.experimental.pallas import tpu as pltpu

WINDOW = 128
HALF = WINDOW // 2
PIX = WINDOW * WINDOW


def _mlp_kernel(tops_ref, lefts_ref, cvals_ref, xs_ref, w1t_ref, b1_ref,
                w2t_ref, b2_ref, w3_ref, b3_ref, out_ref, iidd_ref):
    i = pl.program_id(0)
    top = tops_ref[i]
    left = lefts_ref[i]

    top_base = pl.multiple_of(jnp.minimum((top // 8) * 8, 512 - (WINDOW + 8)), 8)
    off = top - top_base
    slab = xs_ref[:, pl.ds(top_base, WINDOW + 8), :]
    slab = pltpu.roll(slab, ((WINDOW + 8) - off) % (WINDOW + 8), 1)[:, :WINDOW, :]
    crop = pltpu.roll(slab, -left, 2)[:, :, :WINDOW]
    feat = crop.reshape(4, PIX)

    c0 = cvals_ref[i, 0]
    c1 = cvals_ref[i, 1]
    b1c = (b1_ref[0, :] - c0 * w1t_ref[:, 0] - c1 * w1t_ref[:, 1])

    h1 = b1c[:, None] + (w1t_ref[:, 0:1] * feat[0:1, :]
                         + w1t_ref[:, 1:2] * feat[1:2, :]
                         + w1t_ref[:, 2:3] * feat[2:3, :]
                         + w1t_ref[:, 3:4] * feat[3:4, :])
    h1 = jnp.maximum(h1, 0.0)
    h2 = jnp.dot(w2t_ref[...], h1, preferred_element_type=jnp.float32,
                 precision=jax.lax.Precision.HIGHEST)
    h2 = jnp.maximum(h2 + b2_ref[0, :][:, None], 0.0)
    o = jnp.sum(h2 * w3_ref[...], axis=0, keepdims=True)
    out_ref[0, 0, :] = jax.nn.sigmoid(o + b3_ref[0, 0])[0, :]

    p = jax.lax.broadcasted_iota(jnp.int32, (1, PIX), 1)
    iidd_ref[0, 0, 0, :] = jnp.full((PIX,), i, jnp.int32)
    iidd_ref[1, 0, 0, :] = (p // WINDOW + top)[0, :]
    iidd_ref[2, 0, 0, :] = (p % WINDOW + left)[0, :]


@functools.partial(jax.jit, static_argnames=())
def kernel(x, sigma, c, centroids_idx, W1, b1, W2, b2, W3, b3):
    H, W = x.shape[-2:]
    C = c.shape[0]
    E = x.shape[0]

    xs = jnp.concatenate([x, sigma], axis=0)
    tops = jnp.clip(centroids_idx[:, 0], HALF, H - HALF) - HALF
    lefts = jnp.clip(centroids_idx[:, 1], HALF, W - HALF) - HALF

    w1t = W1.T
    w2t = W2.T
    b1r = b1.reshape(1, -1)
    b2r = b2.reshape(1, -1)
    b3r = b3.reshape(1, 1)

    grid = (C,)
    out_shape = (
        jax.ShapeDtypeStruct((C, 1, PIX), jnp.float32),
        jax.ShapeDtypeStruct((3, C, 1, PIX), jnp.int32),
    )
    smem = functools.partial(pl.BlockSpec, memory_space=pltpu.SMEM)
    vmem_full = lambda shp: pl.BlockSpec(shp, lambda i: (0,) * len(shp))
    prob, iidd4 = pl.pallas_call(
        _mlp_kernel,
        grid=grid,
        in_specs=[
            smem((C,), lambda i: (0,)),
            smem((C,), lambda i: (0,)),
            smem((C, 2), lambda i: (0, 0)),
            vmem_full((4, H, W)),
            vmem_full((64, 4)),
            vmem_full((1, 64)),
            vmem_full((64, 64)),
            vmem_full((1, 64)),
            vmem_full((64, 1)),
            vmem_full((1, 1)),
        ],
        out_specs=(
            pl.BlockSpec((1, 1, PIX), lambda i: (i, 0, 0)),
            pl.BlockSpec((3, 1, 1, PIX), lambda i: (0, i, 0, 0)),
        ),
        out_shape=out_shape,
    )(tops, lefts, c, xs, w1t, b1r, w2t, b2r, W3, b3r)

    prob = prob.reshape(C, 1, WINDOW, WINDOW)
    iidd = iidd4.reshape(3, C * PIX)
    return (prob, iidd)


# speedup vs baseline: 71.7650x; 1.2923x over previous
import functools

import jax
import jax.numpy as jnp
from jax.experimental import pallas as pl
from jax.experimental.pallas import tpu as pltpu

WINDOW = 128
HALF = WINDOW // 2
PIX = WINDOW * WINDOW


def _mlp_kernel(tops_ref, lefts_ref, cvals_ref, xs_ref, w1t_ref, b1_ref,
                w2t_ref, b2_ref, w3_ref, b3_ref, out_ref, iidd_ref):
    i = pl.program_id(0)
    top = tops_ref[i]
    left = lefts_ref[i]

    top_base = pl.multiple_of(jnp.minimum((top // 8) * 8, 512 - (WINDOW + 8)), 8)
    off = top - top_base
    slab = xs_ref[:, pl.ds(top_base, WINDOW + 8), :]
    slab = pltpu.roll(slab, ((WINDOW + 8) - off) % (WINDOW + 8), 1)[:, :WINDOW, :]
    crop = pltpu.roll(slab, -left, 2)[:, :, :WINDOW]
    feat = crop.reshape(4, PIX)

    c0 = cvals_ref[i, 0]
    c1 = cvals_ref[i, 1]
    b1c = (b1_ref[0, :] - c0 * w1t_ref[:, 0] - c1 * w1t_ref[:, 1])

    h1 = b1c[:, None] + (w1t_ref[:, 0:1] * feat[0:1, :]
                         + w1t_ref[:, 1:2] * feat[1:2, :]
                         + w1t_ref[:, 2:3] * feat[2:3, :]
                         + w1t_ref[:, 3:4] * feat[3:4, :])
    h1 = jnp.maximum(h1, 0.0)
    h2 = jnp.dot(w2t_ref[...].astype(jnp.bfloat16), h1.astype(jnp.bfloat16),
                 preferred_element_type=jnp.float32)
    h2 = jnp.maximum(h2 + b2_ref[0, :][:, None], 0.0)
    o = jnp.sum(h2 * w3_ref[...], axis=0, keepdims=True)
    out_ref[0, 0, :] = jax.nn.sigmoid(o + b3_ref[0, 0])[0, :]

    p = jax.lax.broadcasted_iota(jnp.int32, (1, PIX), 1)
    iidd_ref[0, 0, 0, :] = jnp.full((PIX,), i, jnp.int32)
    iidd_ref[1, 0, 0, :] = (p // WINDOW + top)[0, :]
    iidd_ref[2, 0, 0, :] = (p % WINDOW + left)[0, :]


@functools.partial(jax.jit, static_argnames=())
def kernel(x, sigma, c, centroids_idx, W1, b1, W2, b2, W3, b3):
    H, W = x.shape[-2:]
    C = c.shape[0]
    E = x.shape[0]

    xs = jnp.concatenate([x, sigma], axis=0)
    tops = jnp.clip(centroids_idx[:, 0], HALF, H - HALF) - HALF
    lefts = jnp.clip(centroids_idx[:, 1], HALF, W - HALF) - HALF

    w1t = W1.T
    w2t = W2.T
    b1r = b1.reshape(1, -1)
    b2r = b2.reshape(1, -1)
    b3r = b3.reshape(1, 1)

    grid = (C,)
    out_shape = (
        jax.ShapeDtypeStruct((C, 1, PIX), jnp.float32),
        jax.ShapeDtypeStruct((3, C, 1, PIX), jnp.int32),
    )
    smem = functools.partial(pl.BlockSpec, memory_space=pltpu.SMEM)
    vmem_full = lambda shp: pl.BlockSpec(shp, lambda i: (0,) * len(shp))
    prob, iidd4 = pl.pallas_call(
        _mlp_kernel,
        grid=grid,
        in_specs=[
            smem((C,), lambda i: (0,)),
            smem((C,), lambda i: (0,)),
            smem((C, 2), lambda i: (0, 0)),
            vmem_full((4, H, W)),
            vmem_full((64, 4)),
            vmem_full((1, 64)),
            vmem_full((64, 64)),
            vmem_full((1, 64)),
            vmem_full((64, 1)),
            vmem_full((1, 1)),
        ],
        out_specs=(
            pl.BlockSpec((1, 1, PIX), lambda i: (i, 0, 0)),
            pl.BlockSpec((3, 1, 1, PIX), lambda i: (0, i, 0, 0)),
        ),
        out_shape=out_shape,
    )(tops, lefts, c, xs, w1t, b1r, w2t, b2r, W3, b3r)

    prob = prob.reshape(C, 1, WINDOW, WINDOW)
    iidd = iidd4.reshape(3, C * PIX)
    return (prob, iidd)


# speedup vs baseline: 111.3544x; 1.5517x over previous
import functools

import jax
import jax.numpy as jnp
from jax.experimental import pallas as pl
from jax.experimental.pallas import tpu as pltpu

WINDOW = 128
HALF = WINDOW // 2
PIX = WINDOW * WINDOW


def _mlp_kernel(tops_ref, lefts_ref, rowb_ref, colb_ref, xs8_ref, w1t_ref,
                b1c_ref, w2t_ref, b2_ref, w3_ref, b3_ref, out_ref, iidd_ref):
    i = pl.program_id(0)
    top = tops_ref[i]
    left = lefts_ref[i]
    base = pl.multiple_of((top // 8) * 8, 8)
    s = top - base

    slab = xs8_ref[s, :, pl.ds(base, WINDOW), :]
    crop = pltpu.roll(slab, (512 - left) % 512, 2)[:, :, :WINDOW]
    feat = crop.reshape(4, PIX)

    h1 = jnp.dot(w1t_ref[...], feat, preferred_element_type=jnp.float32)
    h1 = jnp.maximum(h1 + b1c_ref[0, 0, :][:, None], 0.0)
    h2 = jnp.dot(w2t_ref[...], h1.astype(jnp.bfloat16),
                 preferred_element_type=jnp.float32)
    h2 = jnp.maximum(h2 + b2_ref[...], 0.0)
    o = jnp.sum(h2 * w3_ref[...], axis=0, keepdims=True)
    out_ref[0, 0, :] = (1.0 / (1.0 + jnp.exp(-(o + b3_ref[0, 0]))))[0, :]

    iidd_ref[0, 0, 0, :] = jnp.full((PIX,), i, jnp.int32)
    iidd_ref[1, 0, 0, :] = rowb_ref[0, :] + top
    iidd_ref[2, 0, 0, :] = colb_ref[0, :] + left


@jax.jit
def kernel(x, sigma, c, centroids_idx, W1, b1, W2, b2, W3, b3):
    H, W = x.shape[-2:]
    C = c.shape[0]
    E = x.shape[0]

    xs = jnp.concatenate([x, sigma], axis=0).astype(jnp.bfloat16)
    xs_pad = jnp.pad(xs, ((0, 0), (0, 8), (0, 0)))
    xs8 = jnp.stack([xs_pad[:, sh:sh + H] for sh in range(8)])

    tops = jnp.clip(centroids_idx[:, 0], HALF, H - HALF) - HALF
    lefts = jnp.clip(centroids_idx[:, 1], HALF, W - HALF) - HALF

    w1t = W1.T.astype(jnp.bfloat16)
    w2t = W2.T.astype(jnp.bfloat16)
    b1c = (b1[None, :] - c @ W1[:E]).reshape(C, 1, 64)
    b2col = b2.reshape(-1, 1)
    b3r = b3.reshape(1, 1)
    p = jnp.arange(PIX, dtype=jnp.int32)
    rowb = (p // WINDOW).reshape(1, PIX)
    colb = (p % WINDOW).reshape(1, PIX)

    out_shape = (
        jax.ShapeDtypeStruct((C, 1, PIX), jnp.float32),
        jax.ShapeDtypeStruct((3, C, 1, PIX), jnp.int32),
    )
    smem = functools.partial(pl.BlockSpec, memory_space=pltpu.SMEM)
    vmem_full = lambda shp: pl.BlockSpec(shp, lambda i: (0,) * len(shp))
    prob, iidd4 = pl.pallas_call(
        _mlp_kernel,
        grid=(C,),
        in_specs=[
            smem((C,), lambda i: (0,)),
            smem((C,), lambda i: (0,)),
            vmem_full((1, PIX)),
            vmem_full((1, PIX)),
            vmem_full((8, 4, H, W)),
            vmem_full((64, 4)),
            pl.BlockSpec((1, 1, 64), lambda i: (i, 0, 0)),
            vmem_full((64, 64)),
            vmem_full((64, 1)),
            vmem_full((64, 1)),
            vmem_full((1, 1)),
        ],
        out_specs=(
            pl.BlockSpec((1, 1, PIX), lambda i: (i, 0, 0)),
            pl.BlockSpec((3, 1, 1, PIX), lambda i: (0, i, 0, 0)),
        ),
        out_shape=out_shape,
    )(tops, lefts, rowb, colb, xs8, w1t, b1c, w2t, b2col, W3, b3r)

    prob = prob.reshape(C, 1, WINDOW, WINDOW)
    iidd = iidd4.reshape(3, C * PIX)
    return (prob, iidd)


# speedup vs baseline: 115.1188x; 1.0338x over previous
import functools

import jax
import jax.numpy as jnp
from jax.experimental import pallas as pl
from jax.experimental.pallas import tpu as pltpu

WINDOW = 128
HALF = WINDOW // 2
PIX = WINDOW * WINDOW


def _mlp_kernel(tops_ref, lefts_ref, rowb_ref, colb_ref, x_ref, sig_ref,
                w1t_ref, b1c_ref, w2t_ref, b2_ref, w3_ref, b3_ref,
                out_ref, iidd_ref, xs8):
    i = pl.program_id(0)

    @pl.when(i == 0)
    def _():
        xs_bf = jnp.concatenate([x_ref[...], sig_ref[...]],
                                axis=0).astype(jnp.bfloat16)
        for sh in range(16):
            xs8[sh] = pltpu.roll(xs_bf, (512 - sh) % 512, 1)

    top = tops_ref[i]
    left = lefts_ref[i]
    base = pl.multiple_of((top // 16) * 16, 16)
    s = top - base

    slab = xs8[s, :, pl.ds(base, WINDOW), :]
    crop = pltpu.roll(slab, (512 - left) % 512, 2)[:, :, :WINDOW]
    feat = crop.reshape(4, PIX)

    h1 = jnp.dot(w1t_ref[...], feat, preferred_element_type=jnp.float32)
    h1 = jnp.maximum(h1 + b1c_ref[0, 0, :][:, None], 0).astype(jnp.bfloat16)
    h2 = jnp.dot(w2t_ref[...], h1, preferred_element_type=jnp.float32)
    h2 = jnp.maximum(h2 + b2_ref[...], 0).astype(jnp.bfloat16)
    o = jnp.dot(w3_ref[...], h2, preferred_element_type=jnp.float32)
    out_ref[0, 0, :] = (1.0 / (1.0 + jnp.exp(-(o + b3_ref[0, 0]))))[0, :]

    iidd_ref[0, 0, 0, :] = jnp.full((PIX,), i, jnp.int32)
    iidd_ref[1, 0, 0, :] = rowb_ref[0, :] + top
    iidd_ref[2, 0, 0, :] = colb_ref[0, :] + left


@jax.jit
def kernel(x, sigma, c, centroids_idx, W1, b1, W2, b2, W3, b3):
    H, W = x.shape[-2:]
    C = c.shape[0]
    E = x.shape[0]

    tops = jnp.clip(centroids_idx[:, 0], HALF, H - HALF) - HALF
    lefts = jnp.clip(centroids_idx[:, 1], HALF, W - HALF) - HALF

    w1t = W1.T.astype(jnp.bfloat16)
    w2t = W2.T.astype(jnp.bfloat16)
    w3row = W3.T.astype(jnp.bfloat16)
    b1c = (b1[None, :] - c @ W1[:E]).astype(jnp.bfloat16).reshape(C, 1, 64)
    b2col = b2.astype(jnp.bfloat16).reshape(-1, 1)
    b3r = b3.reshape(1, 1)
    p = jnp.arange(PIX, dtype=jnp.int32)
    rowb = (p // WINDOW).reshape(1, PIX)
    colb = (p % WINDOW).reshape(1, PIX)

    out_shape = (
        jax.ShapeDtypeStruct((C, 1, PIX), jnp.float32),
        jax.ShapeDtypeStruct((3, C, 1, PIX), jnp.int32),
    )
    smem = functools.partial(pl.BlockSpec, memory_space=pltpu.SMEM)
    vmem_full = lambda shp: pl.BlockSpec(shp, lambda i: (0,) * len(shp))
    prob, iidd4 = pl.pallas_call(
        _mlp_kernel,
        grid=(C,),
        in_specs=[
            smem((C,), lambda i: (0,)),
            smem((C,), lambda i: (0,)),
            vmem_full((1, PIX)),
            vmem_full((1, PIX)),
            vmem_full((E, H, W)),
            vmem_full((E, H, W)),
            vmem_full((64, 4)),
            pl.BlockSpec((1, 1, 64), lambda i: (i, 0, 0)),
            vmem_full((64, 64)),
            vmem_full((64, 1)),
            vmem_full((1, 64)),
            vmem_full((1, 1)),
        ],
        out_specs=(
            pl.BlockSpec((1, 1, PIX), lambda i: (i, 0, 0)),
            pl.BlockSpec((3, 1, 1, PIX), lambda i: (0, i, 0, 0)),
        ),
        out_shape=out_shape,
        scratch_shapes=[
            pltpu.VMEM((16, 4, H, W), jnp.bfloat16),
        ],
    )(tops, lefts, rowb, colb, x, sigma, w1t, b1c, w2t, b2col, w3row, b3r)

    prob = prob.reshape(C, 1, WINDOW, WINDOW)
    iidd = iidd4.reshape(3, C * PIX)
    return (prob, iidd)


# speedup vs baseline: 167.9691x; 1.4591x over previous
import functools

import jax
import jax.numpy as jnp
from jax.experimental import pallas as pl
from jax.experimental.pallas import tpu as pltpu

WINDOW = 128
HALF = WINDOW // 2
PIX = WINDOW * WINDOW


def _mlp_kernel(tops_ref, lefts_ref, rowb_ref, colb_ref, x_ref, sig_ref,
                w1t_ref, b1c_ref, w2t_ref, b2_ref, w3_ref, b3_ref,
                out_ref, iidd_ref, xs8):
    i = pl.program_id(0)

    @pl.when(i == 0)
    def _():
        xs_bf = jnp.concatenate([x_ref[...], sig_ref[...]],
                                axis=0).astype(jnp.bfloat16)
        for sh in range(16):
            xs8[sh] = pltpu.roll(xs_bf, (512 - sh) % 512, 1)

    top = tops_ref[i]
    left = lefts_ref[i]
    base = pl.multiple_of((top // 16) * 16, 16)
    s = top - base

    slab = xs8[s, :, pl.ds(base, WINDOW), :]
    crop = pltpu.roll(slab, (512 - left) % 512, 2)[:, :, :WINDOW]
    feat = crop.reshape(4, PIX)

    h1 = jnp.dot(w1t_ref[...], feat, preferred_element_type=jnp.float32)
    h1 = jnp.maximum(h1 + b1c_ref[0, 0, :][:, None], 0).astype(jnp.bfloat16)
    h2 = jnp.dot(w2t_ref[...], h1, preferred_element_type=jnp.float32)
    h2 = jnp.maximum(h2 + b2_ref[...], 0).astype(jnp.bfloat16)
    o = jnp.dot(w3_ref[...], h2, preferred_element_type=jnp.float32)
    out_ref[0, 0, :] = (1.0 / (1.0 + jnp.exp(-(o + b3_ref[0, 0]))))[0, :]

    iidd_ref[0, :] = jnp.full((PIX,), i, jnp.int32)
    iidd_ref[1, :] = rowb_ref[0, :] + top
    iidd_ref[2, :] = colb_ref[0, :] + left


@jax.jit
def kernel(x, sigma, c, centroids_idx, W1, b1, W2, b2, W3, b3):
    H, W = x.shape[-2:]
    C = c.shape[0]
    E = x.shape[0]

    tops = jnp.clip(centroids_idx[:, 0], HALF, H - HALF) - HALF
    lefts = jnp.clip(centroids_idx[:, 1], HALF, W - HALF) - HALF

    w1t = W1.T.astype(jnp.bfloat16)
    w2t = W2.T.astype(jnp.bfloat16)
    w3row = W3.T.astype(jnp.bfloat16)
    b1c = (b1[None, :] - c @ W1[:E]).astype(jnp.bfloat16).reshape(C, 1, 64)
    b2col = b2.astype(jnp.bfloat16).reshape(-1, 1)
    b3r = b3.reshape(1, 1)
    p = jnp.arange(PIX, dtype=jnp.int32)
    rowb = (p // WINDOW).reshape(1, PIX)
    colb = (p % WINDOW).reshape(1, PIX)

    out_shape = (
        jax.ShapeDtypeStruct((C, 1, PIX), jnp.float32),
        jax.ShapeDtypeStruct((3, C * PIX), jnp.int32),
    )
    smem = functools.partial(pl.BlockSpec, memory_space=pltpu.SMEM)
    vmem_full = lambda shp: pl.BlockSpec(shp, lambda i: (0,) * len(shp))
    prob, iidd4 = pl.pallas_call(
        _mlp_kernel,
        grid=(C,),
        in_specs=[
            smem((C,), lambda i: (0,)),
            smem((C,), lambda i: (0,)),
            vmem_full((1, PIX)),
            vmem_full((1, PIX)),
            vmem_full((E, H, W)),
            vmem_full((E, H, W)),
            vmem_full((64, 4)),
            pl.BlockSpec((1, 1, 64), lambda i: (i, 0, 0)),
            vmem_full((64, 64)),
            vmem_full((64, 1)),
            vmem_full((1, 64)),
            vmem_full((1, 1)),
        ],
        out_specs=(
            pl.BlockSpec((1, 1, PIX), lambda i: (i, 0, 0)),
            pl.BlockSpec((3, PIX), lambda i: (0, i)),
        ),
        out_shape=out_shape,
        scratch_shapes=[
            pltpu.VMEM((16, 4, H, W), jnp.bfloat16),
        ],
    )(tops, lefts, rowb, colb, x, sigma, w1t, b1c, w2t, b2col, w3row, b3r)

    prob = prob.reshape(C, 1, WINDOW, WINDOW)
    return (prob, iidd4)


# speedup vs baseline: 168.1206x; 1.0009x over previous
import functools

import jax
import jax.numpy as jnp
from jax.experimental import pallas as pl
from jax.experimental.pallas import tpu as pltpu

WINDOW = 128
HALF = WINDOW // 2
PIX = WINDOW * WINDOW


def _mlp_kernel(tops_ref, lefts_ref, rowb_ref, colb_ref, x_ref, sig_ref,
                w1t_ref, b1c_ref, w2t_ref, b2_ref, w3_ref, b3_ref,
                out_ref, iidd_ref, xs8):
    i = pl.program_id(0)

    @pl.when(i == 0)
    def _():
        xs_bf = jnp.concatenate([x_ref[...], sig_ref[...]],
                                axis=0).astype(jnp.bfloat16)
        for sh in range(16):
            xs8[sh] = pltpu.roll(xs_bf, (512 - sh) % 512, 1)

    top = tops_ref[i]
    left = lefts_ref[i]
    base = pl.multiple_of((top // 16) * 16, 16)
    s = top - base

    slab = xs8[s, :, pl.ds(base, WINDOW), :]
    crop = pltpu.roll(slab, (512 - left) % 512, 2)[:, :, :WINDOW]
    feat = crop.reshape(4, PIX)

    h1 = jnp.dot(w1t_ref[...], feat, preferred_element_type=jnp.float32)
    h1 = jnp.maximum(h1 + b1c_ref[0, 0, :][:, None], 0).astype(jnp.bfloat16)
    h2 = jnp.dot(w2t_ref[...], h1, preferred_element_type=jnp.float32)
    h2 = jnp.maximum(h2 + b2_ref[...], 0).astype(jnp.bfloat16)
    o = jnp.dot(w3_ref[...], h2, preferred_element_type=jnp.float32)
    prob = 1.0 / (1.0 + jnp.exp(-(o + b3_ref[0, 0])))
    out_ref[...] = prob.reshape(WINDOW, WINDOW)

    iidd_ref[0, :] = jnp.full((PIX,), i, jnp.int32)
    iidd_ref[1, :] = rowb_ref[0, :] + top
    iidd_ref[2, :] = colb_ref[0, :] + left


@jax.jit
def kernel(x, sigma, c, centroids_idx, W1, b1, W2, b2, W3, b3):
    H, W = x.shape[-2:]
    C = c.shape[0]
    E = x.shape[0]

    tops = jnp.clip(centroids_idx[:, 0], HALF, H - HALF) - HALF
    lefts = jnp.clip(centroids_idx[:, 1], HALF, W - HALF) - HALF

    w1t = W1.T.astype(jnp.bfloat16)
    w2t = W2.T.astype(jnp.bfloat16)
    w3row = W3.T.astype(jnp.bfloat16)
    b1c = (b1[None, :] - c @ W1[:E]).astype(jnp.bfloat16).reshape(C, 1, 64)
    b2col = b2.astype(jnp.bfloat16).reshape(-1, 1)
    b3r = b3.reshape(1, 1)
    p = jnp.arange(PIX, dtype=jnp.int32)
    rowb = (p // WINDOW).reshape(1, PIX)
    colb = (p % WINDOW).reshape(1, PIX)

    out_shape = (
        jax.ShapeDtypeStruct((C * WINDOW, WINDOW), jnp.float32),
        jax.ShapeDtypeStruct((3, C * PIX), jnp.int32),
    )
    smem = functools.partial(pl.BlockSpec, memory_space=pltpu.SMEM)
    vmem_full = lambda shp: pl.BlockSpec(shp, lambda i: (0,) * len(shp))
    prob, iidd4 = pl.pallas_call(
        _mlp_kernel,
        grid=(C,),
        in_specs=[
            smem((C,), lambda i: (0,)),
            smem((C,), lambda i: (0,)),
            vmem_full((1, PIX)),
            vmem_full((1, PIX)),
            vmem_full((E, H, W)),
            vmem_full((E, H, W)),
            vmem_full((64, 4)),
            pl.BlockSpec((1, 1, 64), lambda i: (i, 0, 0)),
            vmem_full((64, 64)),
            vmem_full((64, 1)),
            vmem_full((1, 64)),
            vmem_full((1, 1)),
        ],
        out_specs=(
            pl.BlockSpec((WINDOW, WINDOW), lambda i: (i, 0)),
            pl.BlockSpec((3, PIX), lambda i: (0, i)),
        ),
        out_shape=out_shape,
        scratch_shapes=[
            pltpu.VMEM((16, 4, H, W), jnp.bfloat16),
        ],
    )(tops, lefts, rowb, colb, x, sigma, w1t, b1c, w2t, b2col, w3row, b3r)

    prob = prob.reshape(C, 1, WINDOW, WINDOW)
    return (prob, iidd4)
